# baseline (speedup 1.0000x reference)
import jax
import jax.numpy as jnp
from jax.experimental import pallas as pl
from jax.experimental.pallas import tpu as pltpu

N = 4096
R = 4
INDIM = 128
OUTDIM = 16

BM = 128
NBUF = 4
NSTEPS = N // BM


def _mrgcn_kernel(x_ref, w2_ref, a_ref, o_ref, xw_ref, abuf, sems):
    def copy_in(step, slot):
        return pltpu.make_async_copy(
            a_ref.at[pl.ds(step * BM, BM), :], abuf.at[slot], sems.at[slot])

    for i in range(NBUF):
        copy_in(i, i).start()

    y = jnp.dot(x_ref[...], w2_ref[...], preferred_element_type=jnp.float32)
    for r in range(R):
        xw_ref[r * N:(r + 1) * N, :] = y[:, r * OUTDIM:(r + 1) * OUTDIM]

    for m in range(NSTEPS):
        slot = m % NBUF
        copy_in(m, slot).wait()
        acc = jnp.dot(abuf[slot], xw_ref[...],
                      preferred_element_type=jnp.float32)
        o_ref[pl.ds(m * BM, BM), :] = jnp.maximum(acc, 0.0)
        nxt = m + NBUF
        if nxt < NSTEPS:
            copy_in(nxt, nxt % NBUF).start()


def kernel(X, A, W):
    W2 = W.reshape(R, INDIM, OUTDIM).transpose(1, 0, 2).reshape(
        INDIM, R * OUTDIM)
    return pl.pallas_call(
        _mrgcn_kernel,
        in_specs=[
            pl.BlockSpec(memory_space=pltpu.VMEM),
            pl.BlockSpec(memory_space=pltpu.VMEM),
            pl.BlockSpec(memory_space=pl.ANY),
        ],
        out_specs=pl.BlockSpec(memory_space=pltpu.VMEM),
        out_shape=jax.ShapeDtypeStruct((N, OUTDIM), jnp.float32),
        scratch_shapes=[
            pltpu.VMEM((R * N, OUTDIM), jnp.float32),
            pltpu.VMEM((NBUF, BM, R * N), jnp.float32),
            pltpu.SemaphoreType.DMA((NBUF,)),
        ],
    )(X, W2, A)
